# baseline (speedup 1.0000x reference)
import functools

import jax
import jax.numpy as jnp
from jax import lax
from jax.experimental import pallas as pl
from jax.experimental.pallas import tpu as pltpu
from jax.experimental.pallas import tpu_sc as plsc

NC = 2
NS = 16
NW = NC * NS
CH = 128

K = 16
D = 128
WD = 32
NG = D // 16


def _wid():
    return lax.axis_index("s") * NC + lax.axis_index("c")


def _sc_mesh():
    return plsc.VectorSubcoreMesh(core_axis_name="c", subcore_axis_name="s")


def _gather_features(idx, node_features, name):
    B = idx.shape[0]
    dt = node_features.dtype
    bpw = B // NW
    nch = max(1, bpw // CH)
    chunk = min(bpw, CH)
    assert bpw % 16 == 0 and (bpw <= CH or bpw % CH == 0)
    assert nch == 1 or nch % 2 == 0

    @functools.partial(
        pl.kernel,
        name=name,
        out_type=jax.ShapeDtypeStruct((B, D), dt),
        mesh=_sc_mesh(),
        scratch_types=[
            pltpu.VMEM((bpw,), jnp.int32),
            pltpu.VMEM((chunk, D), dt),
            pltpu.VMEM((chunk, D), dt),
            pltpu.SemaphoreType.DMA,
            pltpu.SemaphoreType.DMA,
        ],
    )
    def k(idx_h, ft_h, x_o, idx_v, buf0, buf1, s0, s1):
        base = _wid() * bpw
        pltpu.sync_copy(idx_h.at[pl.ds(base, bpw)], idx_v)
        pltpu.async_copy(ft_h.at[idx_v.at[pl.ds(0, chunk)]], buf0, s0)
        if nch == 1:
            pltpu.make_async_copy(ft_h.at[pl.ds(0, chunk)], buf0, s0).wait()
            pltpu.sync_copy(buf0, x_o.at[pl.ds(base, chunk)])
            return

        def body(p, _):
            c0 = 2 * p
            pltpu.async_copy(
                ft_h.at[idx_v.at[pl.ds((c0 + 1) * CH, CH)]], buf1, s1)
            pltpu.make_async_copy(ft_h.at[pl.ds(0, CH)], buf0, s0).wait()
            pltpu.sync_copy(buf0, x_o.at[pl.ds(base + c0 * CH, CH)])

            @pl.when(c0 + 2 < nch)
            def _():
                pltpu.async_copy(
                    ft_h.at[idx_v.at[pl.ds((c0 + 2) * CH, CH)]], buf0, s0)

            pltpu.make_async_copy(ft_h.at[pl.ds(0, CH)], buf1, s1).wait()
            pltpu.sync_copy(buf1, x_o.at[pl.ds(base + (c0 + 1) * CH, CH)])
            return 0

        lax.fori_loop(0, nch // 2, body, 0)

    return k(idx, node_features)


def _sc_hop1_nm(idx, ptf, node_features, amp1):
    OPT = idx.shape[0]
    B2 = OPT // K
    opw = OPT // NW
    rpw = B2 // NW
    nch = opw // CH
    RPC = CH // K
    OB = 16
    nob = nch // OB
    assert opw % CH == 0 and nch % OB == 0 and nob % 2 == 0

    @functools.partial(
        pl.kernel,
        name="sc_hop1_nm",
        out_type=jax.ShapeDtypeStruct((B2, D), jnp.float32),
        mesh=_sc_mesh(),
        compiler_params=pltpu.CompilerParams(needs_layout_passes=False),
        scratch_types=[
            pltpu.VMEM((opw,), jnp.int32),
            pltpu.VMEM((D, D), jnp.float32),
            pltpu.VMEM((opw,), jnp.int32),
            pltpu.VMEM((CH, D), jnp.float32),
            pltpu.VMEM((CH, D), jnp.float32),
            pltpu.VMEM((OB * RPC, D), jnp.float32),
            pltpu.SemaphoreType.DMA,
            pltpu.SemaphoreType.DMA,
        ],
    )
    def k(idx_h, pt_h, ft_h, amp_h, nm_o,
          idx_v, amp_v, ptv, xb0, xb1, nst, sx0, sx1):
        wid = _wid()
        obase = wid * opw
        rbase = wid * rpw
        pltpu.sync_copy(idx_h.at[pl.ds(obase, opw)], idx_v)
        pltpu.sync_copy(pt_h.at[pl.ds(obase, opw)], ptv)
        pltpu.sync_copy(amp_h, amp_v)
        xbs = (xb0, xb1)
        sxs = (sx0, sx1)
        cols = [lax.iota(jnp.int32, 16) + g * 16 for g in range(NG)]
        pltpu.async_copy(ft_h.at[idx_v.at[pl.ds(0, CH)]], xb0, sx0)

        def compute_chunk(xb, c, c2):
            for j2 in range(RPC):
                xb16 = j2 * K
                ob16 = c * CH + xb16
                tv = ptv[pl.ds(ob16, 16)]
                acc = [jnp.zeros((16,), jnp.float32) for _ in range(NG)]
                for kk in range(K):
                    tb = lax.gather(
                        tv, jnp.full((16, 1), kk, jnp.int32),
                        lax.GatherDimensionNumbers(
                            offset_dims=(), collapsed_slice_dims=(0,),
                            start_index_map=(0,)),
                        (1,), mode=lax.GatherScatterMode.PROMISE_IN_BOUNDS)
                    rb = jnp.full((16,), xb16 + kk, jnp.int32)
                    for g in range(NG):
                        av = plsc.load_gather(amp_v, [tb, cols[g]])
                        xv = plsc.load_gather(xb, [rb, cols[g]])
                        acc[g] = acc[g] + av * xv
                row = jnp.full((16,), c2 * RPC + j2, jnp.int32)
                for g in range(NG):
                    plsc.store_scatter(nst, [row, cols[g]], acc[g])

        def chunk_step(c, c2):
            for par in range(2):
                @pl.when(c % 2 == par)
                def _():
                    @pl.when(c + 1 < nch)
                    def _():
                        pltpu.async_copy(
                            ft_h.at[idx_v.at[pl.ds((c + 1) * CH, CH)]],
                            xbs[1 - par], sxs[1 - par])
                    pltpu.make_async_copy(
                        ft_h.at[pl.ds(0, CH)], xbs[par], sxs[par]).wait()
                    compute_chunk(xbs[par], c, c2)

        def ob_body(ob, _):
            def inner(c2, _):
                chunk_step(ob * OB + c2, c2)
                return 0

            lax.fori_loop(0, OB, inner, 0)
            pltpu.sync_copy(
                nst, nm_o.at[pl.ds(rbase + ob * (OB * RPC), OB * RPC)])
            return 0

        lax.fori_loop(0, nob, ob_body, 0)

    return k(idx, ptf, node_features, amp1)


def _amp_tables(wep, wp1, bp1, wp2, bp2):
    def body(wep_ref, wp1_ref, bp1_ref, wp2_ref, bp2_ref, a1_ref, a2_ref):
        we = wep_ref[:]
        a1_ref[:] = jax.nn.sigmoid(
            jnp.dot(we, wp1_ref[:], preferred_element_type=jnp.float32)
            + bp1_ref[:])
        a2_ref[:] = jax.nn.sigmoid(
            jnp.dot(we, wp2_ref[:], preferred_element_type=jnp.float32)
            + bp2_ref[:])

    return pl.pallas_call(
        body,
        out_shape=(jax.ShapeDtypeStruct((D, D), jnp.float32),
                   jax.ShapeDtypeStruct((D, D), jnp.float32)),
    )(wep, wp1, bp1, wp2, bp2)


def _final_layer(x0, pt0, x1r, nm1r, amp1, amp2, wn1, ws1, ba1,
                 wn2, ws2, ba2):
    B = x0.shape[0]
    RB = 512
    grid = B // RB

    def body(x0_ref, pt_ref, x1_ref, nm1_ref, a1_ref, a2_ref,
             wn1_ref, ws1_ref, ba1_ref, wn2_ref, ws2_ref, ba2_ref, out_ref):
        pt = pt_ref[:]
        a1 = a1_ref[:]
        a2 = a2_ref[:]
        wn1w = wn1_ref[:]
        ws1w = ws1_ref[:]
        ba1b = ba1_ref[:]
        acc0 = None
        acc2 = None
        for k in range(K):
            oh = (pt[:, k][:, None] == lax.broadcasted_iota(
                jnp.int32, (RB, D), 1)).astype(jnp.float32)
            x1k = x1_ref[:, k, :]
            amp1k = jnp.dot(oh, a1, preferred_element_type=jnp.float32)
            t0 = amp1k * x1k
            acc0 = t0 if acc0 is None else acc0 + t0
            h1k = jnp.maximum(
                jnp.dot(nm1_ref[:, k, :], wn1w,
                        preferred_element_type=jnp.float32)
                + jnp.dot(x1k, ws1w, preferred_element_type=jnp.float32)
                + ba1b, 0.0)
            amp2k = jnp.dot(oh, a2, preferred_element_type=jnp.float32)
            t2 = amp2k * h1k
            acc2 = t2 if acc2 is None else acc2 + t2
        h0 = jnp.maximum(
            jnp.dot(acc0 * (1.0 / K), wn1w, preferred_element_type=jnp.float32)
            + jnp.dot(x0_ref[:], ws1w, preferred_element_type=jnp.float32)
            + ba1b, 0.0)
        out = (jnp.dot(acc2 * (1.0 / K), wn2_ref[:],
                       preferred_element_type=jnp.float32)
               + jnp.dot(h0, ws2_ref[:], preferred_element_type=jnp.float32)
               + ba2_ref[:])
        nrm = jnp.sqrt(jnp.sum(out * out, axis=1, keepdims=True))
        out_ref[:] = out / jnp.maximum(nrm, 1e-12)

    full = lambda i: (0, 0)
    return pl.pallas_call(
        body,
        grid=(grid,),
        in_specs=[
            pl.BlockSpec((RB, D), lambda i: (i, 0)),
            pl.BlockSpec((RB, K), lambda i: (i, 0)),
            pl.BlockSpec((RB, K, D), lambda i: (i, 0, 0)),
            pl.BlockSpec((RB, K, D), lambda i: (i, 0, 0)),
            pl.BlockSpec((D, D), full),
            pl.BlockSpec((D, D), full),
            pl.BlockSpec((D, D), full),
            pl.BlockSpec((D, D), full),
            pl.BlockSpec((1, D), full),
            pl.BlockSpec((D, D), full),
            pl.BlockSpec((D, D), full),
            pl.BlockSpec((1, D), full),
        ],
        out_specs=pl.BlockSpec((RB, D), lambda i: (i, 0)),
        out_shape=jax.ShapeDtypeStruct((B, D), jnp.float32),
    )(x0, pt0, x1r, nm1r, amp1, amp2, wn1, ws1, ba1, wn2, ws2, ba2)


def kernel(batch_keys, batch_labels, batch_negs, path_types, neigh_ids,
           node_features, walk_embeddings, weight_self_1, weight_neigh_1,
           weight_path_1, bias_path_1, bias_aggregate_1, weight_self_2,
           weight_neigh_2, weight_path_2, bias_path_2, bias_aggregate_2):
    B0 = batch_keys.shape[0]
    ids = jnp.concatenate([batch_keys, batch_labels, batch_negs])
    B = ids.shape[0]

    wep = jnp.zeros((D, WD), jnp.float32).at[:walk_embeddings.shape[0]].set(
        walk_embeddings)
    amp1, amp2 = _amp_tables(
        wep, weight_path_1, bias_path_1.reshape(1, D),
        weight_path_2, bias_path_2.reshape(1, D))

    n1 = jnp.take(neigh_ids, ids, axis=0)
    pt0 = jnp.take(path_types, ids, axis=0)
    x0 = _gather_features(ids, node_features, "sc_gather_x0")
    n1f = n1.reshape(B * K)
    n2 = jnp.take(neigh_ids, n1f, axis=0)
    pt1 = jnp.take(path_types, n1f, axis=0)
    x1 = _gather_features(n1f, node_features, "sc_gather_x1")
    nm1 = _sc_hop1_nm(n2.reshape(B * K * K), pt1.reshape(B * K * K),
                      node_features, amp1 * (1.0 / K))

    out = _final_layer(x0, pt0, x1.reshape(B, K, D), nm1.reshape(B, K, D),
                       amp1, amp2, weight_neigh_1, weight_self_1,
                       bias_aggregate_1.reshape(1, D), weight_neigh_2,
                       weight_self_2, bias_aggregate_2.reshape(1, D))
    return (out[:B0], out[B0:2 * B0], out[2 * B0:])


# speedup vs baseline: 2.3602x; 2.3602x over previous
import functools

import jax
import jax.numpy as jnp
from jax import lax
from jax.experimental import pallas as pl
from jax.experimental.pallas import tpu as pltpu
from jax.experimental.pallas import tpu_sc as plsc

NC = 2
NS = 16
NW = NC * NS
CH = 128

K = 16
D = 128
WD = 32
NG = D // 16


def _wid():
    return lax.axis_index("s") * NC + lax.axis_index("c")


def _sc_mesh():
    return plsc.VectorSubcoreMesh(core_axis_name="c", subcore_axis_name="s")


def _gather_features(idx, node_features, name):
    B = idx.shape[0]
    dt = node_features.dtype
    bpw = B // NW
    nch = max(1, bpw // CH)
    chunk = min(bpw, CH)
    assert bpw % 16 == 0 and (bpw <= CH or bpw % CH == 0)
    assert nch == 1 or nch % 2 == 0

    @functools.partial(
        pl.kernel,
        name=name,
        out_type=jax.ShapeDtypeStruct((B, D), dt),
        mesh=_sc_mesh(),
        scratch_types=[
            pltpu.VMEM((bpw,), jnp.int32),
            pltpu.VMEM((chunk, D), dt),
            pltpu.VMEM((chunk, D), dt),
            pltpu.SemaphoreType.DMA,
            pltpu.SemaphoreType.DMA,
        ],
    )
    def k(idx_h, ft_h, x_o, idx_v, buf0, buf1, s0, s1):
        base = _wid() * bpw
        pltpu.sync_copy(idx_h.at[pl.ds(base, bpw)], idx_v)
        pltpu.async_copy(ft_h.at[idx_v.at[pl.ds(0, chunk)]], buf0, s0)
        if nch == 1:
            pltpu.make_async_copy(ft_h.at[pl.ds(0, chunk)], buf0, s0).wait()
            pltpu.sync_copy(buf0, x_o.at[pl.ds(base, chunk)])
            return

        def body(p, _):
            c0 = 2 * p
            pltpu.async_copy(
                ft_h.at[idx_v.at[pl.ds((c0 + 1) * CH, CH)]], buf1, s1)
            pltpu.make_async_copy(ft_h.at[pl.ds(0, CH)], buf0, s0).wait()
            pltpu.sync_copy(buf0, x_o.at[pl.ds(base + c0 * CH, CH)])

            @pl.when(c0 + 2 < nch)
            def _():
                pltpu.async_copy(
                    ft_h.at[idx_v.at[pl.ds((c0 + 2) * CH, CH)]], buf0, s0)

            pltpu.make_async_copy(ft_h.at[pl.ds(0, CH)], buf1, s1).wait()
            pltpu.sync_copy(buf1, x_o.at[pl.ds(base + (c0 + 1) * CH, CH)])
            return 0

        lax.fori_loop(0, nch // 2, body, 0)

    return k(idx, node_features)


def _sc_hop1_nm(idx, ptf, node_features, amp1):
    OPT = idx.shape[0]
    B2 = OPT // K
    opw = OPT // NW
    rpw = B2 // NW
    nch = opw // CH
    RPC = CH // K
    OB = 16
    nob = nch // OB
    assert opw % CH == 0 and nch % OB == 0 and nob % 2 == 0

    @functools.partial(
        pl.kernel,
        name="sc_hop1_nm",
        out_type=jax.ShapeDtypeStruct((B2, D), jnp.float32),
        mesh=_sc_mesh(),
        compiler_params=pltpu.CompilerParams(needs_layout_passes=False),
        scratch_types=[
            pltpu.VMEM((opw,), jnp.int32),
            pltpu.VMEM((D, D), jnp.float32),
            pltpu.VMEM((opw,), jnp.int32),
            pltpu.VMEM((CH, D), jnp.float32),
            pltpu.VMEM((CH, D), jnp.float32),
            pltpu.VMEM((OB * RPC, D), jnp.float32),
            pltpu.SemaphoreType.DMA,
            pltpu.SemaphoreType.DMA,
        ],
    )
    def k(idx_h, pt_h, ft_h, amp_h, nm_o,
          idx_v, amp_v, ptv, xb0, xb1, nst, sx0, sx1):
        wid = _wid()
        obase = wid * opw
        rbase = wid * rpw
        pltpu.sync_copy(idx_h.at[pl.ds(obase, opw)], idx_v)
        pltpu.sync_copy(pt_h.at[pl.ds(obase, opw)], ptv)
        pltpu.sync_copy(amp_h, amp_v)
        xbs = (xb0, xb1)
        sxs = (sx0, sx1)
        cols = [lax.iota(jnp.int32, 16) + g * 16 for g in range(NG)]
        pltpu.async_copy(ft_h.at[idx_v.at[pl.ds(0, CH)]], xb0, sx0)

        def compute_chunk(xb, c, c2):
            def row_body(j2, _):
                xb16 = j2 * K
                ob16 = c * CH + xb16
                acc = [jnp.zeros((16,), jnp.float32) for _ in range(NG)]
                for kk in range(K):
                    tb = plsc.load_gather(
                        ptv, [jnp.full((16,), ob16 + kk, jnp.int32)])
                    rb = jnp.full((16,), xb16 + kk, jnp.int32)
                    for g in range(NG):
                        av = plsc.load_gather(amp_v, [tb, cols[g]])
                        xv = plsc.load_gather(xb, [rb, cols[g]])
                        acc[g] = acc[g] + av * xv
                row = jnp.full((16,), c2 * RPC + j2, jnp.int32)
                for g in range(NG):
                    plsc.store_scatter(nst, [row, cols[g]], acc[g])
                return 0

            lax.fori_loop(0, RPC, row_body, 0)

        def chunk_step(c, c2):
            for par in range(2):
                @pl.when(c % 2 == par)
                def _():
                    @pl.when(c + 1 < nch)
                    def _():
                        pltpu.async_copy(
                            ft_h.at[idx_v.at[pl.ds((c + 1) * CH, CH)]],
                            xbs[1 - par], sxs[1 - par])
                    pltpu.make_async_copy(
                        ft_h.at[pl.ds(0, CH)], xbs[par], sxs[par]).wait()
                    compute_chunk(xbs[par], c, c2)

        def ob_body(ob, _):
            def inner(c2, _):
                chunk_step(ob * OB + c2, c2)
                return 0

            lax.fori_loop(0, OB, inner, 0)
            pltpu.sync_copy(
                nst, nm_o.at[pl.ds(rbase + ob * (OB * RPC), OB * RPC)])
            return 0

        lax.fori_loop(0, nob, ob_body, 0)

    return k(idx, ptf, node_features, amp1)


def _amp_tables(wep, wp1, bp1, wp2, bp2):
    def body(wep_ref, wp1_ref, bp1_ref, wp2_ref, bp2_ref, a1_ref, a2_ref):
        we = wep_ref[:]
        a1_ref[:] = jax.nn.sigmoid(
            jnp.dot(we, wp1_ref[:], preferred_element_type=jnp.float32)
            + bp1_ref[:])
        a2_ref[:] = jax.nn.sigmoid(
            jnp.dot(we, wp2_ref[:], preferred_element_type=jnp.float32)
            + bp2_ref[:])

    return pl.pallas_call(
        body,
        out_shape=(jax.ShapeDtypeStruct((D, D), jnp.float32),
                   jax.ShapeDtypeStruct((D, D), jnp.float32)),
    )(wep, wp1, bp1, wp2, bp2)


def _final_layer(x0, pt0, x1r, nm1r, amp1, amp2, wn1, ws1, ba1,
                 wn2, ws2, ba2):
    B = x0.shape[0]
    RB = 512
    grid = B // RB

    def body(x0_ref, pt_ref, x1_ref, nm1_ref, a1_ref, a2_ref,
             wn1_ref, ws1_ref, ba1_ref, wn2_ref, ws2_ref, ba2_ref, out_ref):
        pt = pt_ref[:]
        a1 = a1_ref[:]
        a2 = a2_ref[:]
        wn1w = wn1_ref[:]
        ws1w = ws1_ref[:]
        ba1b = ba1_ref[:]
        acc0 = None
        acc2 = None
        for k in range(K):
            oh = (pt[:, k][:, None] == lax.broadcasted_iota(
                jnp.int32, (RB, D), 1)).astype(jnp.float32)
            x1k = x1_ref[:, k, :]
            amp1k = jnp.dot(oh, a1, preferred_element_type=jnp.float32)
            t0 = amp1k * x1k
            acc0 = t0 if acc0 is None else acc0 + t0
            h1k = jnp.maximum(
                jnp.dot(nm1_ref[:, k, :], wn1w,
                        preferred_element_type=jnp.float32)
                + jnp.dot(x1k, ws1w, preferred_element_type=jnp.float32)
                + ba1b, 0.0)
            amp2k = jnp.dot(oh, a2, preferred_element_type=jnp.float32)
            t2 = amp2k * h1k
            acc2 = t2 if acc2 is None else acc2 + t2
        h0 = jnp.maximum(
            jnp.dot(acc0 * (1.0 / K), wn1w, preferred_element_type=jnp.float32)
            + jnp.dot(x0_ref[:], ws1w, preferred_element_type=jnp.float32)
            + ba1b, 0.0)
        out = (jnp.dot(acc2 * (1.0 / K), wn2_ref[:],
                       preferred_element_type=jnp.float32)
               + jnp.dot(h0, ws2_ref[:], preferred_element_type=jnp.float32)
               + ba2_ref[:])
        nrm = jnp.sqrt(jnp.sum(out * out, axis=1, keepdims=True))
        out_ref[:] = out / jnp.maximum(nrm, 1e-12)

    full = lambda i: (0, 0)
    return pl.pallas_call(
        body,
        grid=(grid,),
        in_specs=[
            pl.BlockSpec((RB, D), lambda i: (i, 0)),
            pl.BlockSpec((RB, K), lambda i: (i, 0)),
            pl.BlockSpec((RB, K, D), lambda i: (i, 0, 0)),
            pl.BlockSpec((RB, K, D), lambda i: (i, 0, 0)),
            pl.BlockSpec((D, D), full),
            pl.BlockSpec((D, D), full),
            pl.BlockSpec((D, D), full),
            pl.BlockSpec((D, D), full),
            pl.BlockSpec((1, D), full),
            pl.BlockSpec((D, D), full),
            pl.BlockSpec((D, D), full),
            pl.BlockSpec((1, D), full),
        ],
        out_specs=pl.BlockSpec((RB, D), lambda i: (i, 0)),
        out_shape=jax.ShapeDtypeStruct((B, D), jnp.float32),
    )(x0, pt0, x1r, nm1r, amp1, amp2, wn1, ws1, ba1, wn2, ws2, ba2)


def kernel(batch_keys, batch_labels, batch_negs, path_types, neigh_ids,
           node_features, walk_embeddings, weight_self_1, weight_neigh_1,
           weight_path_1, bias_path_1, bias_aggregate_1, weight_self_2,
           weight_neigh_2, weight_path_2, bias_path_2, bias_aggregate_2):
    B0 = batch_keys.shape[0]
    ids = jnp.concatenate([batch_keys, batch_labels, batch_negs])
    B = ids.shape[0]

    wep = jnp.zeros((D, WD), jnp.float32).at[:walk_embeddings.shape[0]].set(
        walk_embeddings)
    amp1, amp2 = _amp_tables(
        wep, weight_path_1, bias_path_1.reshape(1, D),
        weight_path_2, bias_path_2.reshape(1, D))

    n1 = jnp.take(neigh_ids, ids, axis=0)
    pt0 = jnp.take(path_types, ids, axis=0)
    x0 = _gather_features(ids, node_features, "sc_gather_x0")
    n1f = n1.reshape(B * K)
    n2 = jnp.take(neigh_ids, n1f, axis=0)
    pt1 = jnp.take(path_types, n1f, axis=0)
    x1 = _gather_features(n1f, node_features, "sc_gather_x1")
    nm1 = _sc_hop1_nm(n2.reshape(B * K * K), pt1.reshape(B * K * K),
                      node_features, amp1 * (1.0 / K))

    out = _final_layer(x0, pt0, x1.reshape(B, K, D), nm1.reshape(B, K, D),
                       amp1, amp2, weight_neigh_1, weight_self_1,
                       bias_aggregate_1.reshape(1, D), weight_neigh_2,
                       weight_self_2, bias_aggregate_2.reshape(1, D))
    return (out[:B0], out[B0:2 * B0], out[2 * B0:])


# speedup vs baseline: 2.7560x; 1.1677x over previous
import functools

import jax
import jax.numpy as jnp
from jax import lax
from jax.experimental import pallas as pl
from jax.experimental.pallas import tpu as pltpu
from jax.experimental.pallas import tpu_sc as plsc

NC = 2
NS = 16
NW = NC * NS
CH = 128

K = 16
D = 128
WD = 32
NG = D // 16


def _wid():
    return lax.axis_index("s") * NC + lax.axis_index("c")


def _sc_mesh():
    return plsc.VectorSubcoreMesh(core_axis_name="c", subcore_axis_name="s")


def _gather_features(idx, node_features, name):
    B = idx.shape[0]
    dt = node_features.dtype
    bpw = B // NW
    nch = max(1, bpw // CH)
    chunk = min(bpw, CH)
    assert bpw % 16 == 0 and (bpw <= CH or bpw % CH == 0)
    assert nch == 1 or nch % 2 == 0

    @functools.partial(
        pl.kernel,
        name=name,
        out_type=jax.ShapeDtypeStruct((B, D), dt),
        mesh=_sc_mesh(),
        scratch_types=[
            pltpu.VMEM((bpw,), jnp.int32),
            pltpu.VMEM((chunk, D), dt),
            pltpu.VMEM((chunk, D), dt),
            pltpu.SemaphoreType.DMA,
            pltpu.SemaphoreType.DMA,
        ],
    )
    def k(idx_h, ft_h, x_o, idx_v, buf0, buf1, s0, s1):
        base = _wid() * bpw
        pltpu.sync_copy(idx_h.at[pl.ds(base, bpw)], idx_v)
        pltpu.async_copy(ft_h.at[idx_v.at[pl.ds(0, chunk)]], buf0, s0)
        if nch == 1:
            pltpu.make_async_copy(ft_h.at[pl.ds(0, chunk)], buf0, s0).wait()
            pltpu.sync_copy(buf0, x_o.at[pl.ds(base, chunk)])
            return

        def body(p, _):
            c0 = 2 * p
            pltpu.async_copy(
                ft_h.at[idx_v.at[pl.ds((c0 + 1) * CH, CH)]], buf1, s1)
            pltpu.make_async_copy(ft_h.at[pl.ds(0, CH)], buf0, s0).wait()
            pltpu.sync_copy(buf0, x_o.at[pl.ds(base + c0 * CH, CH)])

            @pl.when(c0 + 2 < nch)
            def _():
                pltpu.async_copy(
                    ft_h.at[idx_v.at[pl.ds((c0 + 2) * CH, CH)]], buf0, s0)

            pltpu.make_async_copy(ft_h.at[pl.ds(0, CH)], buf1, s1).wait()
            pltpu.sync_copy(buf1, x_o.at[pl.ds(base + (c0 + 1) * CH, CH)])
            return 0

        lax.fori_loop(0, nch // 2, body, 0)

    return k(idx, node_features)


def _sc_hop1_nm(idx, ptf, node_features, amp1):
    OPT = idx.shape[0]
    B2 = OPT // K
    opw = OPT // NW
    rpw = B2 // NW
    nch = opw // CH
    RPC = CH // K
    OB = 16
    nob = nch // OB
    assert opw % CH == 0 and nch % OB == 0 and nob % 2 == 0

    @functools.partial(
        pl.kernel,
        name="sc_hop1_nm",
        out_type=jax.ShapeDtypeStruct((B2, D), jnp.float32),
        mesh=_sc_mesh(),
        compiler_params=pltpu.CompilerParams(needs_layout_passes=False),
        scratch_types=[
            pltpu.VMEM((opw,), jnp.int32),
            pltpu.VMEM((D, D), jnp.float32),
            pltpu.VMEM((opw,), jnp.int32),
            pltpu.VMEM((CH, D), jnp.float32),
            pltpu.VMEM((CH, D), jnp.float32),
            pltpu.VMEM((OB * RPC, D), jnp.float32),
            pltpu.SemaphoreType.DMA,
            pltpu.SemaphoreType.DMA,
        ],
    )
    def k(idx_h, pt_h, ft_h, amp_h, nm_o,
          idx_v, amp_v, ptv, xb0, xb1, nst, sx0, sx1):
        wid = _wid()
        obase = wid * opw
        rbase = wid * rpw
        pltpu.sync_copy(idx_h.at[pl.ds(obase, opw)], idx_v)
        pltpu.sync_copy(pt_h.at[pl.ds(obase, opw)], ptv)
        pltpu.sync_copy(amp_h, amp_v)
        xbs = (xb0, xb1)
        sxs = (sx0, sx1)
        cols = [lax.iota(jnp.int32, 16) + g * 16 for g in range(NG)]
        pltpu.async_copy(ft_h.at[idx_v.at[pl.ds(0, CH)]], xb0, sx0)

        def compute_chunk(xb, c, c2):
            def row_body(j2, _):
                xb16 = j2 * K
                ob16 = c * CH + xb16
                acc = [jnp.zeros((16,), jnp.float32) for _ in range(NG)]
                for kk in range(K):
                    tb = plsc.load_gather(
                        ptv, [jnp.full((16,), ob16 + kk, jnp.int32)])
                    for g in range(NG):
                        av = plsc.load_gather(amp_v, [tb, cols[g]])
                        xv = xb[xb16 + kk, pl.ds(g * 16, 16)]
                        acc[g] = acc[g] + av * xv
                for g in range(NG):
                    nst[c2 * RPC + j2, pl.ds(g * 16, 16)] = (
                        acc[g] * (1.0 / K))
                return 0

            lax.fori_loop(0, RPC, row_body, 0)

        def chunk_step(c, c2):
            for par in range(2):
                @pl.when(c % 2 == par)
                def _():
                    @pl.when(c + 1 < nch)
                    def _():
                        pltpu.async_copy(
                            ft_h.at[idx_v.at[pl.ds((c + 1) * CH, CH)]],
                            xbs[1 - par], sxs[1 - par])
                    pltpu.make_async_copy(
                        ft_h.at[pl.ds(0, CH)], xbs[par], sxs[par]).wait()
                    compute_chunk(xbs[par], c, c2)

        def ob_body(ob, _):
            def inner(c2, _):
                chunk_step(ob * OB + c2, c2)
                return 0

            lax.fori_loop(0, OB, inner, 0)
            pltpu.sync_copy(
                nst, nm_o.at[pl.ds(rbase + ob * (OB * RPC), OB * RPC)])
            return 0

        lax.fori_loop(0, nob, ob_body, 0)

    return k(idx, ptf, node_features, amp1)


def _amp_tables(wep, wp1, bp1, wp2, bp2):
    def body(wep_ref, wp1_ref, bp1_ref, wp2_ref, bp2_ref, a1_ref, a2_ref):
        we = wep_ref[:]
        a1_ref[:] = jax.nn.sigmoid(
            jnp.dot(we, wp1_ref[:], preferred_element_type=jnp.float32)
            + bp1_ref[:])
        a2_ref[:] = jax.nn.sigmoid(
            jnp.dot(we, wp2_ref[:], preferred_element_type=jnp.float32)
            + bp2_ref[:])

    return pl.pallas_call(
        body,
        out_shape=(jax.ShapeDtypeStruct((D, D), jnp.float32),
                   jax.ShapeDtypeStruct((D, D), jnp.float32)),
    )(wep, wp1, bp1, wp2, bp2)


def _final_layer(x0, pt0, x1r, nm1r, amp1, amp2, wn1, ws1, ba1,
                 wn2, ws2, ba2):
    B = x0.shape[0]
    RB = 512
    grid = B // RB

    def body(x0_ref, pt_ref, x1_ref, nm1_ref, a1_ref, a2_ref,
             wn1_ref, ws1_ref, ba1_ref, wn2_ref, ws2_ref, ba2_ref, out_ref):
        pt = pt_ref[:]
        a1 = a1_ref[:]
        a2 = a2_ref[:]
        wn1w = wn1_ref[:]
        ws1w = ws1_ref[:]
        ba1b = ba1_ref[:]
        acc0 = None
        acc2 = None
        for k in range(K):
            oh = (pt[:, k][:, None] == lax.broadcasted_iota(
                jnp.int32, (RB, D), 1)).astype(jnp.float32)
            x1k = x1_ref[:, k, :]
            amp1k = jnp.dot(oh, a1, preferred_element_type=jnp.float32)
            t0 = amp1k * x1k
            acc0 = t0 if acc0 is None else acc0 + t0
            h1k = jnp.maximum(
                jnp.dot(nm1_ref[:, k, :], wn1w,
                        preferred_element_type=jnp.float32)
                + jnp.dot(x1k, ws1w, preferred_element_type=jnp.float32)
                + ba1b, 0.0)
            amp2k = jnp.dot(oh, a2, preferred_element_type=jnp.float32)
            t2 = amp2k * h1k
            acc2 = t2 if acc2 is None else acc2 + t2
        h0 = jnp.maximum(
            jnp.dot(acc0 * (1.0 / K), wn1w, preferred_element_type=jnp.float32)
            + jnp.dot(x0_ref[:], ws1w, preferred_element_type=jnp.float32)
            + ba1b, 0.0)
        out = (jnp.dot(acc2 * (1.0 / K), wn2_ref[:],
                       preferred_element_type=jnp.float32)
               + jnp.dot(h0, ws2_ref[:], preferred_element_type=jnp.float32)
               + ba2_ref[:])
        nrm = jnp.sqrt(jnp.sum(out * out, axis=1, keepdims=True))
        out_ref[:] = out / jnp.maximum(nrm, 1e-12)

    full = lambda i: (0, 0)
    return pl.pallas_call(
        body,
        grid=(grid,),
        in_specs=[
            pl.BlockSpec((RB, D), lambda i: (i, 0)),
            pl.BlockSpec((RB, K), lambda i: (i, 0)),
            pl.BlockSpec((RB, K, D), lambda i: (i, 0, 0)),
            pl.BlockSpec((RB, K, D), lambda i: (i, 0, 0)),
            pl.BlockSpec((D, D), full),
            pl.BlockSpec((D, D), full),
            pl.BlockSpec((D, D), full),
            pl.BlockSpec((D, D), full),
            pl.BlockSpec((1, D), full),
            pl.BlockSpec((D, D), full),
            pl.BlockSpec((D, D), full),
            pl.BlockSpec((1, D), full),
        ],
        out_specs=pl.BlockSpec((RB, D), lambda i: (i, 0)),
        out_shape=jax.ShapeDtypeStruct((B, D), jnp.float32),
    )(x0, pt0, x1r, nm1r, amp1, amp2, wn1, ws1, ba1, wn2, ws2, ba2)


def kernel(batch_keys, batch_labels, batch_negs, path_types, neigh_ids,
           node_features, walk_embeddings, weight_self_1, weight_neigh_1,
           weight_path_1, bias_path_1, bias_aggregate_1, weight_self_2,
           weight_neigh_2, weight_path_2, bias_path_2, bias_aggregate_2):
    B0 = batch_keys.shape[0]
    ids = jnp.concatenate([batch_keys, batch_labels, batch_negs])
    B = ids.shape[0]

    wep = jnp.zeros((D, WD), jnp.float32).at[:walk_embeddings.shape[0]].set(
        walk_embeddings)
    amp1, amp2 = _amp_tables(
        wep, weight_path_1, bias_path_1.reshape(1, D),
        weight_path_2, bias_path_2.reshape(1, D))

    n1 = jnp.take(neigh_ids, ids, axis=0)
    pt0 = jnp.take(path_types, ids, axis=0)
    x0 = _gather_features(ids, node_features, "sc_gather_x0")
    n1f = n1.reshape(B * K)
    n2 = jnp.take(neigh_ids, n1f, axis=0)
    pt1 = jnp.take(path_types, n1f, axis=0)
    x1 = _gather_features(n1f, node_features, "sc_gather_x1")
    nm1 = _sc_hop1_nm(n2.reshape(B * K * K), pt1.reshape(B * K * K),
                      node_features, amp1)

    out = _final_layer(x0, pt0, x1.reshape(B, K, D), nm1.reshape(B, K, D),
                       amp1, amp2, weight_neigh_1, weight_self_1,
                       bias_aggregate_1.reshape(1, D), weight_neigh_2,
                       weight_self_2, bias_aggregate_2.reshape(1, D))
    return (out[:B0], out[B0:2 * B0], out[2 * B0:])
